# baseline (speedup 1.0000x reference)
import functools

import jax
import jax.numpy as jnp
from jax import lax
from jax.experimental import pallas as pl
from jax.experimental.pallas import tpu as pltpu
from jax.experimental.pallas import tpu_sc as plsc

B = 4096
ED = 64
NHIST = 50
NWISH = 20
NTAGS = 10
NLANG = 51
NCORES = 2
NSUB = 16
NW = NCORES * NSUB
ROWS_W = B // NW
HC = 8
NCH = ROWS_W // HC
GSUB = 80
NH_BOOK = 500224
NH_AUTH = 50176
NH_TAG = 512

_f32 = jnp.float32


HBLK = 512


def _tr_body(in0_ref, in1_ref, out_ref):
    out_ref[:, 0:ED] = in0_ref[...].T
    out_ref[:, ED:2 * ED] = in1_ref[...].T


def _transpose_table(tbl, n_rows):
    g = -(-n_rows // (2 * HBLK))
    out = pl.pallas_call(
        _tr_body,
        grid=(g,),
        in_specs=[pl.BlockSpec((ED, HBLK), lambda j: (0, j)),
                  pl.BlockSpec((ED, HBLK), lambda j: (0, j + g))],
        out_specs=pl.BlockSpec((HBLK, 2 * ED), lambda j: (j, 0)),
        out_shape=jax.ShapeDtypeStruct((g * HBLK, 2 * ED), _f32),
    )(tbl.T, tbl.T)
    return out.reshape(2 * g * HBLK, ED), g * HBLK


def _sc_body(hist_hbm, wish_hbm, tags_hbm, bid_hbm, auth_hbm,
             book2d, auth2d, tag2d,
             u0_hbm, ip_hbm,
             idx_h, idx_w, idx_t, idx_b, idx_a,
             rows_h, rows_w, rows_t, rows_b, rows_a,
             out_u, out_i, sem):
    wid = lax.axis_index("s") * NCORES + lax.axis_index("c")
    rbase = wid * ROWS_W

    pltpu.sync_copy(hist_hbm.at[pl.ds(rbase * NHIST, ROWS_W * NHIST)], idx_h)
    pltpu.sync_copy(wish_hbm.at[pl.ds(rbase * NWISH, ROWS_W * NWISH)], idx_w)
    pltpu.sync_copy(tags_hbm.at[pl.ds(rbase * NTAGS, ROWS_W * NTAGS)], idx_t)
    pltpu.sync_copy(bid_hbm.at[pl.ds(rbase, ROWS_W)], idx_b)
    pltpu.sync_copy(auth_hbm.at[pl.ds(rbase, ROWS_W)], idx_a)

    def remap(idxref, n, nh):
        for s in range(0, n, 16):
            v = idxref[pl.ds(s, 16)]
            two = v + v
            idxref[pl.ds(s, 16)] = jnp.where(v < nh, two, two - (2 * nh - 1))

    remap(idx_h, ROWS_W * NHIST, NH_BOOK)
    remap(idx_w, ROWS_W * NWISH, NH_BOOK)
    remap(idx_t, ROWS_W * NTAGS, NH_TAG)
    remap(idx_b, ROWS_W, NH_BOOK)
    remap(idx_a, ROWS_W, NH_AUTH)

    def chunk(c, carry):
        hoff = c * (HC * NHIST)
        woff = c * (HC * NWISH)
        toff = c * (HC * NTAGS)
        boff = c * HC
        cps = []
        for k in range(HC * NHIST // GSUB):
            cps.append(pltpu.async_copy(
                book2d.at[idx_h.at[pl.ds(hoff + k * GSUB, GSUB)]],
                rows_h.at[pl.ds(k * GSUB, GSUB)], sem))
        for k in range(HC * NWISH // GSUB):
            cps.append(pltpu.async_copy(
                book2d.at[idx_w.at[pl.ds(woff + k * GSUB, GSUB)]],
                rows_w.at[pl.ds(k * GSUB, GSUB)], sem))
        cps.append(pltpu.async_copy(
            tag2d.at[idx_t.at[pl.ds(toff, HC * NTAGS)]], rows_t, sem))
        cps.append(pltpu.async_copy(
            book2d.at[idx_b.at[pl.ds(boff, HC)]], rows_b, sem))
        cps.append(pltpu.async_copy(
            auth2d.at[idx_a.at[pl.ds(boff, HC)]], rows_a, sem))
        for cp in cps:
            cp.wait()

        for r in range(HC):
            for g in range(ED // 16):
                s = pl.ds(g * 16, 16)
                acch = rows_h[r * NHIST, s]
                for j in range(1, NHIST):
                    acch = acch + rows_h[r * NHIST + j, s]
                accw = rows_w[r * NWISH, s]
                for j in range(1, NWISH):
                    accw = accw + rows_w[r * NWISH + j, s]
                out_u[r, s] = acch * (1.0 / NHIST) + accw * (1.0 / NWISH)
                acct = rows_t[r * NTAGS, s]
                for j in range(1, NTAGS):
                    acct = acct + rows_t[r * NTAGS + j, s]
                out_i[r, s] = (rows_b[r, s] + rows_a[r, s]
                               + acct * (1.0 / NTAGS))

        pltpu.sync_copy(out_u, u0_hbm.at[pl.ds(rbase + c * HC, HC)])
        pltpu.sync_copy(out_i, ip_hbm.at[pl.ds(rbase + c * HC, HC)])
        return carry

    lax.fori_loop(0, NCH, chunk, 0)


_sc_gather_pool = functools.partial(
    pl.kernel,
    out_type=(jax.ShapeDtypeStruct((B, ED), _f32),
              jax.ShapeDtypeStruct((B, ED), _f32)),
    mesh=plsc.VectorSubcoreMesh(core_axis_name="c", subcore_axis_name="s"),
    scratch_types=[
        pltpu.VMEM((ROWS_W * NHIST,), jnp.int32),
        pltpu.VMEM((ROWS_W * NWISH,), jnp.int32),
        pltpu.VMEM((ROWS_W * NTAGS,), jnp.int32),
        pltpu.VMEM((ROWS_W,), jnp.int32),
        pltpu.VMEM((ROWS_W,), jnp.int32),
        pltpu.VMEM((HC * NHIST, ED), _f32),
        pltpu.VMEM((HC * NWISH, ED), _f32),
        pltpu.VMEM((HC * NTAGS, ED), _f32),
        pltpu.VMEM((HC, ED), _f32),
        pltpu.VMEM((HC, ED), _f32),
        pltpu.VMEM((HC, ED), _f32),
        pltpu.VMEM((HC, ED), _f32),
        pltpu.SemaphoreType.DMA,
    ],
    compiler_params=pltpu.CompilerParams(use_tc_tiling_on_sc=False),
)(_sc_body)


def _tc_body(u0, ipart, dense, lang, lemb, w1, b1, w2, b2,
             wu1, bu1, wu2, bu2, wu3, bu3, out):
    uh = jnp.maximum(
        jnp.dot(u0[...], wu1[...], preferred_element_type=_f32) + bu1[...], 0.0)
    uh = jnp.maximum(
        jnp.dot(uh, wu2[...], preferred_element_type=_f32) + bu2[...], 0.0)
    u_emb = jnp.dot(uh, wu3[...], preferred_element_type=_f32) + bu3[...]
    d = dense[...]
    w1v = w1[...]
    dh = (d[:, 0:1] * w1v[0:1, :] + d[:, 1:2] * w1v[1:2, :]
          + d[:, 2:3] * w1v[2:3, :] + b1[...])
    dh = jnp.maximum(dh, 0.0)
    d_e = jnp.dot(dh, w2[...], preferred_element_type=_f32) + b2[...]
    onehot = (lang[...] == lax.broadcasted_iota(jnp.int32, (1, NLANG), 1))
    l_e = jnp.dot(onehot.astype(_f32), lemb[...],
                  preferred_element_type=_f32)
    i_emb = ipart[...] + d_e + l_e
    out[...] = jnp.sum(u_emb * i_emb, axis=1, keepdims=True)


def kernel(hist_ids, wish_ids, bid, auth, lang, tags, dense,
           book_emb, auth_emb, lang_emb, tag_emb,
           W1, b1, W2, b2, Wu1, bu1, Wu2, bu2, Wu3, bu3):
    book2d, nh_book = _transpose_table(book_emb, 1000000)
    auth2d, nh_auth = _transpose_table(auth_emb, 100000)
    tag2d, nh_tag = _transpose_table(tag_emb, 1000)
    assert (nh_book, nh_auth, nh_tag) == (NH_BOOK, NH_AUTH, NH_TAG)
    u0, ipart = _sc_gather_pool(
        hist_ids.reshape(-1), wish_ids.reshape(-1), tags.reshape(-1),
        bid, auth, book2d, auth2d, tag2d)
    out = pl.pallas_call(
        _tc_body,
        out_shape=jax.ShapeDtypeStruct((B, 1), _f32),
    )(u0, ipart, dense, lang.reshape(B, 1), lang_emb,
      W1, b1.reshape(1, -1), W2, b2.reshape(1, -1),
      Wu1, bu1.reshape(1, -1), Wu2, bu2.reshape(1, -1),
      Wu3, bu3.reshape(1, -1))
    return out


# speedup vs baseline: 1.7025x; 1.7025x over previous
import functools

import jax
import jax.numpy as jnp
from jax import lax
from jax.experimental import pallas as pl
from jax.experimental.pallas import tpu as pltpu
from jax.experimental.pallas import tpu_sc as plsc

B = 4096
ED = 64
NHIST = 50
NWISH = 20
NTAGS = 10
NLANG = 51
NCORES = 2
NSUB = 16
NW = NCORES * NSUB
ROWS_W = B // NW
HC = 8
NCH = ROWS_W // HC
GSUB = 80
TBLK = 4096
THALF = TBLK // 2

_f32 = jnp.float32


_TDIMS = (((0,), (0,)), ((), ()))


def _tr_body(in_ref, eye_ref, out_ref):
    e = eye_ref[...]
    t0 = lax.dot_general(in_ref[:, 0:THALF], e, _TDIMS,
                         preferred_element_type=_f32)
    t1 = lax.dot_general(in_ref[:, THALF:TBLK], e, _TDIMS,
                         preferred_element_type=_f32)
    out_ref[...] = jnp.concatenate([t0, t1], axis=1)


def _transpose_table(tbl, n_rows):
    g = -(-n_rows // TBLK)
    out = pl.pallas_call(
        _tr_body,
        grid=(g,),
        in_specs=[pl.BlockSpec((ED, TBLK), lambda j: (0, j)),
                  pl.BlockSpec((ED, ED), lambda j: (0, 0))],
        out_specs=pl.BlockSpec((THALF, 2 * ED), lambda j: (j, 0)),
        out_shape=jax.ShapeDtypeStruct((g * THALF, 2 * ED), _f32),
    )(tbl.T, jnp.eye(ED, dtype=_f32))
    return out.reshape(g * TBLK, ED)


def _sc_body(hist_hbm, wish_hbm, tags_hbm, bid_hbm, auth_hbm,
             book2d, auth2d, tag2d,
             u0_hbm, ip_hbm,
             idx_h, idx_w, idx_t, idx_b, idx_a,
             rows_h, rows_w, rows_t, rows_b, rows_a,
             out_u, out_i, sem):
    wid = lax.axis_index("s") * NCORES + lax.axis_index("c")
    rbase = wid * ROWS_W

    pltpu.sync_copy(hist_hbm.at[pl.ds(rbase * NHIST, ROWS_W * NHIST)], idx_h)
    pltpu.sync_copy(wish_hbm.at[pl.ds(rbase * NWISH, ROWS_W * NWISH)], idx_w)
    pltpu.sync_copy(tags_hbm.at[pl.ds(rbase * NTAGS, ROWS_W * NTAGS)], idx_t)
    pltpu.sync_copy(bid_hbm.at[pl.ds(rbase, ROWS_W)], idx_b)
    pltpu.sync_copy(auth_hbm.at[pl.ds(rbase, ROWS_W)], idx_a)

    def remap(idxref, n):
        for s in range(0, n, 16):
            r = idxref[pl.ds(s, 16)]
            v = r & (TBLK - 1)
            two = v + v
            lin = jnp.where(v < THALF, two, two - (TBLK - 1))
            idxref[pl.ds(s, 16)] = (r - v) + lin

    remap(idx_h, ROWS_W * NHIST)
    remap(idx_w, ROWS_W * NWISH)
    remap(idx_t, ROWS_W * NTAGS)
    remap(idx_b, ROWS_W)
    remap(idx_a, ROWS_W)

    def chunk(c, carry):
        hoff = c * (HC * NHIST)
        woff = c * (HC * NWISH)
        toff = c * (HC * NTAGS)
        boff = c * HC
        cps = []
        for k in range(HC * NHIST // GSUB):
            cps.append(pltpu.async_copy(
                book2d.at[idx_h.at[pl.ds(hoff + k * GSUB, GSUB)]],
                rows_h.at[pl.ds(k * GSUB, GSUB)], sem))
        for k in range(HC * NWISH // GSUB):
            cps.append(pltpu.async_copy(
                book2d.at[idx_w.at[pl.ds(woff + k * GSUB, GSUB)]],
                rows_w.at[pl.ds(k * GSUB, GSUB)], sem))
        cps.append(pltpu.async_copy(
            tag2d.at[idx_t.at[pl.ds(toff, HC * NTAGS)]], rows_t, sem))
        cps.append(pltpu.async_copy(
            book2d.at[idx_b.at[pl.ds(boff, HC)]], rows_b, sem))
        cps.append(pltpu.async_copy(
            auth2d.at[idx_a.at[pl.ds(boff, HC)]], rows_a, sem))
        for cp in cps:
            cp.wait()

        for r in range(HC):
            for g in range(ED // 16):
                s = pl.ds(g * 16, 16)
                acch = rows_h[r * NHIST, s]
                for j in range(1, NHIST):
                    acch = acch + rows_h[r * NHIST + j, s]
                accw = rows_w[r * NWISH, s]
                for j in range(1, NWISH):
                    accw = accw + rows_w[r * NWISH + j, s]
                out_u[r, s] = acch * (1.0 / NHIST) + accw * (1.0 / NWISH)
                acct = rows_t[r * NTAGS, s]
                for j in range(1, NTAGS):
                    acct = acct + rows_t[r * NTAGS + j, s]
                out_i[r, s] = (rows_b[r, s] + rows_a[r, s]
                               + acct * (1.0 / NTAGS))

        pltpu.sync_copy(out_u, u0_hbm.at[pl.ds(rbase + c * HC, HC)])
        pltpu.sync_copy(out_i, ip_hbm.at[pl.ds(rbase + c * HC, HC)])
        return carry

    lax.fori_loop(0, NCH, chunk, 0)


_sc_gather_pool = functools.partial(
    pl.kernel,
    out_type=(jax.ShapeDtypeStruct((B, ED), _f32),
              jax.ShapeDtypeStruct((B, ED), _f32)),
    mesh=plsc.VectorSubcoreMesh(core_axis_name="c", subcore_axis_name="s"),
    scratch_types=[
        pltpu.VMEM((ROWS_W * NHIST,), jnp.int32),
        pltpu.VMEM((ROWS_W * NWISH,), jnp.int32),
        pltpu.VMEM((ROWS_W * NTAGS,), jnp.int32),
        pltpu.VMEM((ROWS_W,), jnp.int32),
        pltpu.VMEM((ROWS_W,), jnp.int32),
        pltpu.VMEM((HC * NHIST, ED), _f32),
        pltpu.VMEM((HC * NWISH, ED), _f32),
        pltpu.VMEM((HC * NTAGS, ED), _f32),
        pltpu.VMEM((HC, ED), _f32),
        pltpu.VMEM((HC, ED), _f32),
        pltpu.VMEM((HC, ED), _f32),
        pltpu.VMEM((HC, ED), _f32),
        pltpu.SemaphoreType.DMA,
    ],
    compiler_params=pltpu.CompilerParams(use_tc_tiling_on_sc=False),
)(_sc_body)


def _tc_body(u0, ipart, dense, lang, lemb, w1, b1, w2, b2,
             wu1, bu1, wu2, bu2, wu3, bu3, out):
    uh = jnp.maximum(
        jnp.dot(u0[...], wu1[...], preferred_element_type=_f32) + bu1[...], 0.0)
    uh = jnp.maximum(
        jnp.dot(uh, wu2[...], preferred_element_type=_f32) + bu2[...], 0.0)
    u_emb = jnp.dot(uh, wu3[...], preferred_element_type=_f32) + bu3[...]
    d = dense[...]
    w1v = w1[...]
    dh = (d[:, 0:1] * w1v[0:1, :] + d[:, 1:2] * w1v[1:2, :]
          + d[:, 2:3] * w1v[2:3, :] + b1[...])
    dh = jnp.maximum(dh, 0.0)
    d_e = jnp.dot(dh, w2[...], preferred_element_type=_f32) + b2[...]
    onehot = (lang[...] == lax.broadcasted_iota(jnp.int32, (1, NLANG), 1))
    l_e = jnp.dot(onehot.astype(_f32), lemb[...],
                  preferred_element_type=_f32)
    i_emb = ipart[...] + d_e + l_e
    out[...] = jnp.sum(u_emb * i_emb, axis=1, keepdims=True)


def kernel(hist_ids, wish_ids, bid, auth, lang, tags, dense,
           book_emb, auth_emb, lang_emb, tag_emb,
           W1, b1, W2, b2, Wu1, bu1, Wu2, bu2, Wu3, bu3):
    book2d = _transpose_table(book_emb, 1000000)
    auth2d = _transpose_table(auth_emb, 100000)
    tag2d = _transpose_table(tag_emb, 1000)
    u0, ipart = _sc_gather_pool(
        hist_ids.reshape(-1), wish_ids.reshape(-1), tags.reshape(-1),
        bid, auth, book2d, auth2d, tag2d)
    out = pl.pallas_call(
        _tc_body,
        out_shape=jax.ShapeDtypeStruct((B, 1), _f32),
    )(u0, ipart, dense, lang.reshape(B, 1), lang_emb,
      W1, b1.reshape(1, -1), W2, b2.reshape(1, -1),
      Wu1, bu1.reshape(1, -1), Wu2, bu2.reshape(1, -1),
      Wu3, bu3.reshape(1, -1))
    return out


# speedup vs baseline: 1.9650x; 1.1542x over previous
import functools

import jax
import jax.numpy as jnp
from jax import lax
from jax.experimental import pallas as pl
from jax.experimental.pallas import tpu as pltpu
from jax.experimental.pallas import tpu_sc as plsc

B = 4096
ED = 64
NHIST = 50
NWISH = 20
NTAGS = 10
NLANG = 51
NCORES = 2
NSUB = 16
NW = NCORES * NSUB
ROWS_W = B // NW
HC = 8
NCH = ROWS_W // HC
GSUB = 80
TBLK = 8192
THALF = TBLK // 2

_f32 = jnp.float32


_TDIMS = (((0,), (0,)), ((), ()))


def _tr_body(in_ref, eye_ref, out_ref):
    e = eye_ref[...]
    out_ref[:, 0:ED] = lax.dot_general(
        in_ref[:, 0:THALF], e, _TDIMS, preferred_element_type=_f32)
    out_ref[:, ED:2 * ED] = lax.dot_general(
        in_ref[:, THALF:TBLK], e, _TDIMS, preferred_element_type=_f32)


def _transpose_table(tbl, n_rows):
    g = -(-n_rows // TBLK)
    out = pl.pallas_call(
        _tr_body,
        grid=(g,),
        in_specs=[pl.BlockSpec((ED, TBLK), lambda j: (0, j)),
                  pl.BlockSpec((ED, ED), lambda j: (0, 0))],
        out_specs=pl.BlockSpec((THALF, 2 * ED), lambda j: (j, 0)),
        out_shape=jax.ShapeDtypeStruct((g * THALF, 2 * ED), _f32),
    )(tbl.T, jnp.eye(ED, dtype=_f32))
    return out.reshape(g * TBLK, ED)


def _sc_body(hist_hbm, wish_hbm, tags_hbm, bid_hbm, auth_hbm,
             book2d, auth2d, tag2d,
             u0_hbm, ip_hbm,
             idx_h, idx_w, idx_t, idx_b, idx_a,
             rows_h, rows_w, rows_t, rows_b, rows_a,
             out_u, out_i, sem):
    wid = lax.axis_index("s") * NCORES + lax.axis_index("c")
    rbase = wid * ROWS_W

    pltpu.sync_copy(hist_hbm.at[pl.ds(rbase * NHIST, ROWS_W * NHIST)], idx_h)
    pltpu.sync_copy(wish_hbm.at[pl.ds(rbase * NWISH, ROWS_W * NWISH)], idx_w)
    pltpu.sync_copy(tags_hbm.at[pl.ds(rbase * NTAGS, ROWS_W * NTAGS)], idx_t)
    pltpu.sync_copy(bid_hbm.at[pl.ds(rbase, ROWS_W)], idx_b)
    pltpu.sync_copy(auth_hbm.at[pl.ds(rbase, ROWS_W)], idx_a)

    def remap(idxref, n):
        for s in range(0, n, 16):
            r = idxref[pl.ds(s, 16)]
            v = r & (TBLK - 1)
            two = v + v
            lin = jnp.where(v < THALF, two, two - (TBLK - 1))
            idxref[pl.ds(s, 16)] = (r - v) + lin

    remap(idx_h, ROWS_W * NHIST)
    remap(idx_w, ROWS_W * NWISH)
    remap(idx_t, ROWS_W * NTAGS)
    remap(idx_b, ROWS_W)
    remap(idx_a, ROWS_W)

    def chunk(c, carry):
        hoff = c * (HC * NHIST)
        woff = c * (HC * NWISH)
        toff = c * (HC * NTAGS)
        boff = c * HC
        cps = []
        for k in range(HC * NHIST // GSUB):
            cps.append(pltpu.async_copy(
                book2d.at[idx_h.at[pl.ds(hoff + k * GSUB, GSUB)]],
                rows_h.at[pl.ds(k * GSUB, GSUB)], sem))
        for k in range(HC * NWISH // GSUB):
            cps.append(pltpu.async_copy(
                book2d.at[idx_w.at[pl.ds(woff + k * GSUB, GSUB)]],
                rows_w.at[pl.ds(k * GSUB, GSUB)], sem))
        cps.append(pltpu.async_copy(
            tag2d.at[idx_t.at[pl.ds(toff, HC * NTAGS)]], rows_t, sem))
        cps.append(pltpu.async_copy(
            book2d.at[idx_b.at[pl.ds(boff, HC)]], rows_b, sem))
        cps.append(pltpu.async_copy(
            auth2d.at[idx_a.at[pl.ds(boff, HC)]], rows_a, sem))
        for cp in cps:
            cp.wait()

        for r in range(HC):
            for g in range(ED // 16):
                s = pl.ds(g * 16, 16)
                acch = rows_h[r * NHIST, s]
                for j in range(1, NHIST):
                    acch = acch + rows_h[r * NHIST + j, s]
                accw = rows_w[r * NWISH, s]
                for j in range(1, NWISH):
                    accw = accw + rows_w[r * NWISH + j, s]
                out_u[r, s] = acch * (1.0 / NHIST) + accw * (1.0 / NWISH)
                acct = rows_t[r * NTAGS, s]
                for j in range(1, NTAGS):
                    acct = acct + rows_t[r * NTAGS + j, s]
                out_i[r, s] = (rows_b[r, s] + rows_a[r, s]
                               + acct * (1.0 / NTAGS))

        pltpu.sync_copy(out_u, u0_hbm.at[pl.ds(rbase + c * HC, HC)])
        pltpu.sync_copy(out_i, ip_hbm.at[pl.ds(rbase + c * HC, HC)])
        return carry

    lax.fori_loop(0, NCH, chunk, 0)


_sc_gather_pool = functools.partial(
    pl.kernel,
    out_type=(jax.ShapeDtypeStruct((B, ED), _f32),
              jax.ShapeDtypeStruct((B, ED), _f32)),
    mesh=plsc.VectorSubcoreMesh(core_axis_name="c", subcore_axis_name="s"),
    scratch_types=[
        pltpu.VMEM((ROWS_W * NHIST,), jnp.int32),
        pltpu.VMEM((ROWS_W * NWISH,), jnp.int32),
        pltpu.VMEM((ROWS_W * NTAGS,), jnp.int32),
        pltpu.VMEM((ROWS_W,), jnp.int32),
        pltpu.VMEM((ROWS_W,), jnp.int32),
        pltpu.VMEM((HC * NHIST, ED), _f32),
        pltpu.VMEM((HC * NWISH, ED), _f32),
        pltpu.VMEM((HC * NTAGS, ED), _f32),
        pltpu.VMEM((HC, ED), _f32),
        pltpu.VMEM((HC, ED), _f32),
        pltpu.VMEM((HC, ED), _f32),
        pltpu.VMEM((HC, ED), _f32),
        pltpu.SemaphoreType.DMA,
    ],
    compiler_params=pltpu.CompilerParams(use_tc_tiling_on_sc=False),
)(_sc_body)


def _tc_body(u0, ipart, dense, lang, lemb, w1, b1, w2, b2,
             wu1, bu1, wu2, bu2, wu3, bu3, out):
    uh = jnp.maximum(
        jnp.dot(u0[...], wu1[...], preferred_element_type=_f32) + bu1[...], 0.0)
    uh = jnp.maximum(
        jnp.dot(uh, wu2[...], preferred_element_type=_f32) + bu2[...], 0.0)
    u_emb = jnp.dot(uh, wu3[...], preferred_element_type=_f32) + bu3[...]
    d = dense[...]
    w1v = w1[...]
    dh = (d[:, 0:1] * w1v[0:1, :] + d[:, 1:2] * w1v[1:2, :]
          + d[:, 2:3] * w1v[2:3, :] + b1[...])
    dh = jnp.maximum(dh, 0.0)
    d_e = jnp.dot(dh, w2[...], preferred_element_type=_f32) + b2[...]
    onehot = (lang[...] == lax.broadcasted_iota(jnp.int32, (1, NLANG), 1))
    l_e = jnp.dot(onehot.astype(_f32), lemb[...],
                  preferred_element_type=_f32)
    i_emb = ipart[...] + d_e + l_e
    out[...] = jnp.sum(u_emb * i_emb, axis=1, keepdims=True)


def kernel(hist_ids, wish_ids, bid, auth, lang, tags, dense,
           book_emb, auth_emb, lang_emb, tag_emb,
           W1, b1, W2, b2, Wu1, bu1, Wu2, bu2, Wu3, bu3):
    book2d = _transpose_table(book_emb, 1000000)
    auth2d = _transpose_table(auth_emb, 100000)
    tag2d = _transpose_table(tag_emb, 1000)
    u0, ipart = _sc_gather_pool(
        hist_ids.reshape(-1), wish_ids.reshape(-1), tags.reshape(-1),
        bid, auth, book2d, auth2d, tag2d)
    out = pl.pallas_call(
        _tc_body,
        out_shape=jax.ShapeDtypeStruct((B, 1), _f32),
    )(u0, ipart, dense, lang.reshape(B, 1), lang_emb,
      W1, b1.reshape(1, -1), W2, b2.reshape(1, -1),
      Wu1, bu1.reshape(1, -1), Wu2, bu2.reshape(1, -1),
      Wu3, bu3.reshape(1, -1))
    return out
